# baseline (device time: 66231 ns/iter reference)
import numpy as np
import jax
import jax.numpy as jnp
from jax import lax
from jax.experimental import pallas as pl
from jax.experimental.pallas import tpu as pltpu

N_DEV = 8
N_PEERS = N_DEV - 1
N_QUARTERS = 4


def _mask_bias(sq: int, skv: int, block: int) -> np.ndarray:
    qb = (np.arange(sq) // block)[:, None]
    kb = (np.arange(skv) // block)[None, :]
    mask = (qb == kb) | (kb == 0) | ((qb + kb) % 3 == 0)
    return np.where(mask, 0.0, -1e9).astype(np.float32)


def _bitrev3(c: int) -> int:
    return ((c & 1) << 2) | (c & 2) | ((c >> 2) & 1)


def kernel(x, Wq, K_ext, V_ext, Wo):
    B, Sq, E = x.shape
    _, Skv, H, Dh = K_ext.shape
    HD = H * Dh
    CH = Sq // N_DEV
    QR = Sq // N_QUARTERS

    my = lax.axis_index("i")
    Wq_i = lax.dynamic_slice(Wq, (0, my * HD), (E, HD))
    Wo_i = lax.dynamic_slice(Wo, (my * HD, 0), (HD, E))
    Kt = K_ext.transpose(0, 2, 1, 3)
    Vt = V_ext.transpose(0, 2, 1, 3)
    bias = jnp.asarray(_mask_bias(Sq, Skv, 64))

    def body(x_ref, wq_ref, kt_ref, vt_ref, wo_ref, bias_ref,
             out_ref, acc_ref, rs_s, rs_r,
             rs_send_sems, rs_recv_sems, ag_send_sems, ag_recv_sems):
        my_pos = lax.axis_index("i")

        barrier = pltpu.get_barrier_semaphore()
        for k in range(1, N_DEV):
            pl.semaphore_signal(
                barrier, inc=1,
                device_id=(my_pos ^ k,), device_id_type=pl.DeviceIdType.MESH,
            )
        pl.semaphore_wait(barrier, N_PEERS)

        wq = wq_ref[...].astype(jnp.bfloat16)
        wo = wo_ref[...].astype(jnp.bfloat16)

        rs_sends = []
        for qt in range(N_QUARTERS):
            rows = slice(qt * QR, (qt + 1) * QR)
            b_mat = bias_ref[rows, :]
            for b in range(B):
                xq = x_ref[b, rows, :].astype(jnp.bfloat16)
                q_all = jnp.dot(xq, wq, preferred_element_type=jnp.float32)
                ctxs = []
                for h in range(H):
                    q = q_all[:, h * Dh:(h + 1) * Dh].astype(jnp.bfloat16)
                    k = kt_ref[b, h].astype(jnp.bfloat16)
                    s = lax.dot_general(
                        q, k, (((1,), (1,)), ((), ())),
                        preferred_element_type=jnp.float32,
                    )
                    s = s * 0.125 + b_mat
                    m = jnp.max(s, axis=1, keepdims=True)
                    p = jnp.exp(s - m)
                    p = p / jnp.sum(p, axis=1, keepdims=True)
                    v = vt_ref[b, h].astype(jnp.bfloat16)
                    ctxs.append(jnp.dot(p.astype(jnp.bfloat16), v,
                                        preferred_element_type=jnp.float32))
                ctx = jnp.concatenate(ctxs, axis=1).astype(jnp.bfloat16)
                acc_ref[b, rows, :] = jnp.dot(
                    ctx, wo, preferred_element_type=jnp.float32)
            for c in range(qt * 2, qt * 2 + 2):
                owner = _bitrev3(c)
                rs_s[c] = acc_ref[:, c * CH:(c + 1) * CH, :].astype(
                    jnp.bfloat16)

                @pl.when(owner != my_pos)
                def _(c=c, owner=owner):
                    slot = (my_pos ^ owner) - 1
                    pltpu.make_async_remote_copy(
                        src_ref=rs_s.at[c],
                        dst_ref=rs_r.at[slot],
                        send_sem=rs_send_sems.at[c],
                        recv_sem=rs_recv_sems.at[slot],
                        device_id=(owner,),
                        device_id_type=pl.DeviceIdType.MESH,
                    ).start()

                rs_sends.append((c, owner))

        for k in range(1, N_DEV):
            recv = pltpu.make_async_remote_copy(
                src_ref=rs_s.at[0],
                dst_ref=rs_r.at[k - 1],
                send_sem=rs_send_sems.at[0],
                recv_sem=rs_recv_sems.at[k - 1],
                device_id=(my_pos ^ k,),
                device_id_type=pl.DeviceIdType.MESH,
            )
            recv.wait_recv()

        my_off = pl.multiple_of(
            CH * (4 * (my_pos & 1) + 2 * ((my_pos >> 1) & 1)
                  + ((my_pos >> 2) & 1)), CH)
        total = acc_ref[:, pl.ds(my_off, CH), :]
        for k in range(1, N_DEV):
            total = total + rs_r[k - 1].astype(jnp.float32)
        out_ref[:, pl.ds(my_off, CH), :] = total.astype(jnp.bfloat16)

        ag = []
        for k in range(1, N_DEV):
            rdma = pltpu.make_async_remote_copy(
                src_ref=out_ref.at[:, pl.ds(my_off, CH), :],
                dst_ref=out_ref.at[:, pl.ds(my_off, CH), :],
                send_sem=ag_send_sems.at[k - 1],
                recv_sem=ag_recv_sems.at[k - 1],
                device_id=(my_pos ^ k,),
                device_id_type=pl.DeviceIdType.MESH,
            )
            rdma.start()
            ag.append(rdma)

        for c, owner in rs_sends:
            @pl.when(owner != my_pos)
            def _(c=c, owner=owner):
                pltpu.make_async_remote_copy(
                    src_ref=rs_s.at[c],
                    dst_ref=rs_r.at[0],
                    send_sem=rs_send_sems.at[c],
                    recv_sem=rs_recv_sems.at[0],
                    device_id=(owner,),
                    device_id_type=pl.DeviceIdType.MESH,
                ).wait_send()

        for rdma in ag:
            rdma.wait_send()
        for rdma in ag:
            rdma.wait_recv()

    return pl.pallas_call(
        body,
        out_shape=jax.ShapeDtypeStruct((B, Sq, E), jnp.bfloat16),
        in_specs=[pl.BlockSpec(memory_space=pltpu.VMEM)] * 6,
        out_specs=pl.BlockSpec(memory_space=pltpu.VMEM),
        scratch_shapes=[
            pltpu.VMEM((B, Sq, E), jnp.float32),
            pltpu.VMEM((N_DEV, B, CH, E), jnp.bfloat16),
            pltpu.VMEM((N_PEERS, B, CH, E), jnp.bfloat16),
            pltpu.SemaphoreType.DMA((N_DEV,)),
            pltpu.SemaphoreType.DMA((N_PEERS,)),
            pltpu.SemaphoreType.DMA((N_PEERS,)),
            pltpu.SemaphoreType.DMA((N_PEERS,)),
        ],
        compiler_params=pltpu.CompilerParams(collective_id=0),
    )(x, Wq_i, Kt, Vt, Wo_i, bias)
